# baseline (speedup 1.0000x reference)
import functools

import jax
import jax.numpy as jnp
from jax import lax
from jax.experimental import pallas as pl
from jax.experimental.pallas import tpu as pltpu
from jax.experimental.pallas import tpu_sc as plsc

_B = 16384
_D = 64
_EPS = 1e-3

_NC, _NS = 2, 16
_NW = _NC * _NS
_BPW = _B // _NW

@functools.cache
def _make_sc_gather():
    mesh = plsc.VectorSubcoreMesh(core_axis_name="c", subcore_axis_name="s")

    @functools.partial(
        pl.kernel,
        mesh=mesh,
        out_type=[
            jax.ShapeDtypeStruct((_B, _D), jnp.float32),
            jax.ShapeDtypeStruct((_B, _D), jnp.float32),
        ],
        scratch_types=[
            pltpu.VMEM((_BPW,), jnp.int32),
            pltpu.VMEM((_BPW,), jnp.int32),
            pltpu.VMEM((_BPW, _D), jnp.float32),
            pltpu.VMEM((_BPW, _D), jnp.float32),
            pltpu.SemaphoreType.DMA,
            pltpu.SemaphoreType.DMA,
        ],
        compiler_params=pltpu.CompilerParams(use_tc_tiling_on_sc=False),
    )
    def sc_gather(uidx_hbm, iidx_hbm, utab_hbm, itab_hbm, uout_hbm, iout_hbm,
                  uidx_v, iidx_v, urows_v, irows_v, usem, isem):
        wid = lax.axis_index("s") * _NC + lax.axis_index("c")
        base = wid * _BPW
        pltpu.sync_copy(uidx_hbm.at[pl.ds(base, _BPW)], uidx_v)
        pltpu.sync_copy(iidx_hbm.at[pl.ds(base, _BPW)], iidx_v)
        cu = pltpu.async_copy(utab_hbm.at[uidx_v], urows_v, usem)
        ci = pltpu.async_copy(itab_hbm.at[iidx_v], irows_v, isem)
        cu.wait()
        pltpu.sync_copy(urows_v, uout_hbm.at[pl.ds(base, _BPW)])
        ci.wait()
        pltpu.sync_copy(irows_v, iout_hbm.at[pl.ds(base, _BPW)])

    return sc_gather


_BB = 2048


def _mlp_body(u_ref, it_ref, w1_ref, b1_ref, s1_ref, t1_ref,
              w2_ref, b2_ref, s2_ref, t2_ref,
              w3_ref, b3_ref, s3_ref, t3_ref,
              wo_ref, bo_ref, out_ref):
    x = jnp.concatenate([u_ref[...], it_ref[...]], axis=1)
    h = jnp.dot(x, w1_ref[...], preferred_element_type=jnp.float32)
    h = jnp.maximum(h + b1_ref[...], 0.0) * s1_ref[...] + t1_ref[...]
    h = jnp.dot(h, w2_ref[...], preferred_element_type=jnp.float32)
    h = jnp.maximum(h + b2_ref[...], 0.0) * s2_ref[...] + t2_ref[...]
    h = jnp.dot(h, w3_ref[...], preferred_element_type=jnp.float32)
    h = jnp.maximum(h + b3_ref[...], 0.0) * s3_ref[...] + t3_ref[...]
    out_ref[...] = (jnp.sum(h * wo_ref[...], axis=1, keepdims=True)
                    + bo_ref[...])


def _full(shape):
    return pl.BlockSpec(shape, lambda i: (0, 0))


_mlp_call = pl.pallas_call(
    _mlp_body,
    grid=(_B // _BB,),
    in_specs=[
        pl.BlockSpec((_BB, _D), lambda i: (i, 0)),
        pl.BlockSpec((_BB, _D), lambda i: (i, 0)),
        _full((2 * _D, 128)), _full((1, 128)), _full((1, 128)), _full((1, 128)),
        _full((128, 64)), _full((1, 64)), _full((1, 64)), _full((1, 64)),
        _full((64, 32)), _full((1, 32)), _full((1, 32)), _full((1, 32)),
        _full((1, 32)), _full((1, 1)),
    ],
    out_specs=pl.BlockSpec((_BB, 1), lambda i: (i, 0)),
    out_shape=jax.ShapeDtypeStruct((_B, 1), jnp.float32),
)


def kernel(inputs, user_emb, item_emb, W1, b1, g1, be1, m1, v1,
           W2, b2, g2, be2, m2, v2, W3, b3, g3, be3, m3, v3, Wout, bout):
    uidx = inputs[:, 0]
    iidx = inputs[:, 1]
    u, it = _make_sc_gather()(uidx, iidx, user_emb, item_emb)

    def fold(g, be, m, v):
        s = g * lax.rsqrt(v + _EPS)
        return (s[None, :], (be - m * s)[None, :])

    s1, t1 = fold(g1, be1, m1, v1)
    s2, t2 = fold(g2, be2, m2, v2)
    s3, t3 = fold(g3, be3, m3, v3)
    return _mlp_call(u, it,
                     W1, b1[None, :], s1, t1,
                     W2, b2[None, :], s2, t2,
                     W3, b3[None, :], s3, t3,
                     Wout.T, bout[None, :])


# speedup vs baseline: 2.3802x; 2.3802x over previous
import functools

import jax
import jax.numpy as jnp
from jax import lax
from jax.experimental import pallas as pl
from jax.experimental.pallas import tpu as pltpu
from jax.experimental.pallas import tpu_sc as plsc

_B = 16384
_D = 64
_V = 1000000
_EPS = 1e-3

_NC, _NS = 2, 16
_NW = _NC * _NS
_BPW = _B // _NW
_NBUF = 4


@functools.cache
def _make_sc_gather():
    mesh = plsc.VectorSubcoreMesh(core_axis_name="c", subcore_axis_name="s")

    @functools.partial(
        pl.kernel,
        mesh=mesh,
        out_type=jax.ShapeDtypeStruct((_B, 2 * _D), jnp.float32),
        scratch_types=[
            pltpu.VMEM((_BPW + 16,), jnp.int32),
            pltpu.VMEM((_NBUF, 8, 8, 128), jnp.float32),
            pltpu.VMEM((_BPW, 2 * _D), jnp.float32),
        ] + [pltpu.SemaphoreType.DMA] * _NBUF,
        compiler_params=pltpu.CompilerParams(needs_layout_passes=False),
    )
    def sc_gather(uidx_hbm, iidx_hbm, utab_hbm, itab_hbm, x_hbm,
                  idx_v, ring, staging, *sems):
        wid = lax.axis_index("s") * _NC + lax.axis_index("c")
        base = wid * _BPW

        lane = lax.iota(jnp.int32, 16)
        d_vecs = [lane + (16 * k) for k in range(4)]
        a_vecs = [d >> 3 for d in d_vecs]
        j_vecs = [d & 7 for d in d_vecs]

        def col_slice(tab, v):
            col = pl.multiple_of((v >> 7) << 7, 128)
            return tab.at[:, :, pl.ds(col, 128)]

        def fetch(tab, slot, v):
            pltpu.async_copy(col_slice(tab, v), ring.at[slot], sems[slot])

        def drain(tab, slot):
            pltpu.make_async_copy(
                col_slice(tab, 0), ring.at[slot], sems[slot]).wait()

        def extract(slot, row, m, col_off):
            for k in range(4):
                vals = plsc.load_gather(
                    ring,
                    [jnp.full((16,), slot, jnp.int32), a_vecs[k], j_vecs[k],
                     jnp.full((16,), m, jnp.int32)])
                plsc.store_scatter(
                    staging,
                    [jnp.full((16,), row, jnp.int32), d_vecs[k] + col_off],
                    vals)

        def sread(e):
            vec = idx_v[pl.ds(e, 16)]
            return jnp.sum(jnp.where(lane == 0, vec, 0))

        for tab, sidx_hbm, col_off in (
                (utab_hbm, uidx_hbm, 0), (itab_hbm, iidx_hbm, _D)):
            pltpu.sync_copy(sidx_hbm.at[pl.ds(base, _BPW)],
                            idx_v.at[pl.ds(0, _BPW)])
            for slot in range(_NBUF):
                fetch(tab, slot, sread(slot))

            def chunk(g, _, tab=tab, col_off=col_off):
                for slot in range(_NBUF):
                    row = g * _NBUF + slot
                    v = sread(row)
                    drain(tab, slot)
                    extract(slot, row, v & 127, col_off)

                    @pl.when(row + _NBUF < _BPW)
                    def _():
                        fetch(tab, slot, sread(row + _NBUF))
                return ()

            lax.fori_loop(0, _BPW // _NBUF, chunk, (), unroll=False)

        pltpu.sync_copy(staging, x_hbm.at[pl.ds(base, _BPW), :])

    return sc_gather


_BB = 2048


def _mlp_body(x_ref, w1_ref, b1_ref, s1_ref, t1_ref,
              w2_ref, b2_ref, s2_ref, t2_ref,
              w3_ref, b3_ref, s3_ref, t3_ref,
              wo_ref, bo_ref, out_ref):
    h = jnp.dot(x_ref[...], w1_ref[...], preferred_element_type=jnp.float32)
    h = jnp.maximum(h + b1_ref[...], 0.0) * s1_ref[...] + t1_ref[...]
    h = jnp.dot(h, w2_ref[...], preferred_element_type=jnp.float32)
    h = jnp.maximum(h + b2_ref[...], 0.0) * s2_ref[...] + t2_ref[...]
    h = jnp.dot(h, w3_ref[...], preferred_element_type=jnp.float32)
    h = jnp.maximum(h + b3_ref[...], 0.0) * s3_ref[...] + t3_ref[...]
    out_ref[...] = (jnp.sum(h * wo_ref[...], axis=1, keepdims=True)
                    + bo_ref[...])


def _full(shape):
    return pl.BlockSpec(shape, lambda i: (0, 0))


_mlp_call = pl.pallas_call(
    _mlp_body,
    grid=(_B // _BB,),
    in_specs=[
        pl.BlockSpec((_BB, 2 * _D), lambda i: (i, 0)),
        _full((2 * _D, 128)), _full((1, 128)), _full((1, 128)), _full((1, 128)),
        _full((128, 64)), _full((1, 64)), _full((1, 64)), _full((1, 64)),
        _full((64, 32)), _full((1, 32)), _full((1, 32)), _full((1, 32)),
        _full((1, 32)), _full((1, 1)),
    ],
    out_specs=pl.BlockSpec((_BB, 1), lambda i: (i, 0)),
    out_shape=jax.ShapeDtypeStruct((_B, 1), jnp.float32),
)


def kernel(inputs, user_emb, item_emb, W1, b1, g1, be1, m1, v1,
           W2, b2, g2, be2, m2, v2, W3, b3, g3, be3, m3, v3, Wout, bout):
    uidx = inputs[:, 0]
    iidx = inputs[:, 1]
    utab3 = user_emb.T.reshape(8, 8, _V)
    itab3 = item_emb.T.reshape(8, 8, _V)
    x = _make_sc_gather()(uidx, iidx, utab3, itab3)

    def fold(g, be, m, v):
        s = g * lax.rsqrt(v + _EPS)
        return (s[None, :], (be - m * s)[None, :])

    s1, t1 = fold(g1, be1, m1, v1)
    s2, t2 = fold(g2, be2, m2, v2)
    s3, t3 = fold(g3, be3, m3, v3)
    return _mlp_call(x,
                     W1, b1[None, :], s1, t1,
                     W2, b2[None, :], s2, t2,
                     W3, b3[None, :], s3, t3,
                     Wout.T, bout[None, :])


# speedup vs baseline: 3.0145x; 1.2665x over previous
import functools

import jax
import jax.numpy as jnp
from jax import lax
from jax.experimental import pallas as pl
from jax.experimental.pallas import tpu as pltpu
from jax.experimental.pallas import tpu_sc as plsc

_B = 16384
_D = 64
_V = 1000000
_EPS = 1e-3

_S = 8192
_NC, _NS = 2, 16
_NW = _NC * _NS
_BPW = _S // _NW
_NBUF = 4

_CH = 128
_BTC = _B - _S


@functools.cache
def _make_sc_gather():
    mesh = plsc.VectorSubcoreMesh(core_axis_name="c", subcore_axis_name="s")

    @functools.partial(
        pl.kernel,
        mesh=mesh,
        out_type=jax.ShapeDtypeStruct((2 * _D, _S), jnp.float32),
        scratch_types=[
            pltpu.VMEM((_BPW + 16,), jnp.int32),
            pltpu.VMEM((_NBUF, 8, 8, 128), jnp.float32),
            pltpu.VMEM((2 * _D, _BPW), jnp.float32),
        ] + [pltpu.SemaphoreType.DMA] * _NBUF,
        compiler_params=pltpu.CompilerParams(needs_layout_passes=False),
    )
    def sc_gather(uidx_hbm, iidx_hbm, utab_hbm, itab_hbm, xT_hbm,
                  idx_v, ring, staging, *sems):
        wid = lax.axis_index("s") * _NC + lax.axis_index("c")
        base = wid * _BPW

        lane = lax.iota(jnp.int32, 16)
        d_vecs = [lane + (16 * k) for k in range(4)]
        a_vecs = [d >> 3 for d in d_vecs]
        j_vecs = [d & 7 for d in d_vecs]

        def col_slice(tab, v):
            col = pl.multiple_of((v >> 7) << 7, 128)
            return tab.at[:, :, pl.ds(col, 128)]

        def fetch(tab, slot, v):
            pltpu.async_copy(col_slice(tab, v), ring.at[slot], sems[slot])

        def drain(tab, slot):
            pltpu.make_async_copy(
                col_slice(tab, 0), ring.at[slot], sems[slot]).wait()

        def extract(slot, row, m, col_off):
            for k in range(4):
                vals = plsc.load_gather(
                    ring,
                    [jnp.full((16,), slot, jnp.int32), a_vecs[k], j_vecs[k],
                     jnp.full((16,), m, jnp.int32)])
                plsc.store_scatter(
                    staging,
                    [d_vecs[k] + col_off, jnp.full((16,), row, jnp.int32)],
                    vals)

        def sread(e):
            vec = idx_v[pl.ds(e, 16)]
            return jnp.sum(jnp.where(lane == 0, vec, 0))

        for tab, sidx_hbm, col_off in (
                (utab_hbm, uidx_hbm, 0), (itab_hbm, iidx_hbm, _D)):
            pltpu.sync_copy(sidx_hbm.at[pl.ds(base, _BPW)],
                            idx_v.at[pl.ds(0, _BPW)])
            for slot in range(_NBUF):
                fetch(tab, slot, sread(slot))

            def chunk(g, _, tab=tab, col_off=col_off):
                for slot in range(_NBUF):
                    row = g * _NBUF + slot
                    v = sread(row)
                    drain(tab, slot)
                    extract(slot, row, v & 127, col_off)

                    @pl.when(row + _NBUF < _BPW)
                    def _():
                        fetch(tab, slot, sread(row + _NBUF))
                return ()

            lax.fori_loop(0, _BPW // _NBUF, chunk, (), unroll=False)

        pltpu.sync_copy(staging, xT_hbm.at[:, pl.ds(base, _BPW)])

    return sc_gather


def _mlp_from_xT(xT, w1, b1, s1, t1, w2, b2, s2, t2, w3, b3, s3, t3, wo, bo):
    h = lax.dot_general(xT, w1, (((0,), (0,)), ((), ())),
                        preferred_element_type=jnp.float32)
    h = jnp.maximum(h + b1, 0.0) * s1 + t1
    h = jnp.dot(h, w2, preferred_element_type=jnp.float32)
    h = jnp.maximum(h + b2, 0.0) * s2 + t2
    h = jnp.dot(h, w3, preferred_element_type=jnp.float32)
    h = jnp.maximum(h + b3, 0.0) * s3 + t3
    return jnp.sum(h * wo, axis=1, keepdims=True) + bo


_BB = 2048


def _mlp_body(x_ref, w1_ref, b1_ref, s1_ref, t1_ref,
              w2_ref, b2_ref, s2_ref, t2_ref,
              w3_ref, b3_ref, s3_ref, t3_ref,
              wo_ref, bo_ref, out_ref):
    out_ref[...] = _mlp_from_xT(
        x_ref[...], w1_ref[...], b1_ref[...], s1_ref[...], t1_ref[...],
        w2_ref[...], b2_ref[...], s2_ref[...], t2_ref[...],
        w3_ref[...], b3_ref[...], s3_ref[...], t3_ref[...],
        wo_ref[...], bo_ref[...])


def _full(shape):
    return pl.BlockSpec(shape, lambda i: tuple(0 for _ in shape))


_mlp_call = pl.pallas_call(
    _mlp_body,
    grid=(_S // _BB,),
    in_specs=[
        pl.BlockSpec((2 * _D, _BB), lambda i: (0, i)),
        _full((2 * _D, 128)), _full((1, 128)), _full((1, 128)), _full((1, 128)),
        _full((128, 64)), _full((1, 64)), _full((1, 64)), _full((1, 64)),
        _full((64, 32)), _full((1, 32)), _full((1, 32)), _full((1, 32)),
        _full((1, 32)), _full((1, 1)),
    ],
    out_specs=pl.BlockSpec((_BB, 1), lambda i: (i, 0)),
    out_shape=jax.ShapeDtypeStruct((_S, 1), jnp.float32),
)


def _tc_body(cu_ref, ci_ref, mu_ref, mi_ref,
             utab_ref, itab_ref,
             w1_ref, b1_ref, s1_ref, t1_ref,
             w2_ref, b2_ref, s2_ref, t2_ref,
             w3_ref, b3_ref, s3_ref, t3_ref,
             wo_ref, bo_ref, out_ref,
             ubuf, ibuf, usem, isem):
    g = pl.program_id(0)
    nch = pl.num_programs(0)

    def issue(slot, gg):
        for e in range(_CH):
            cu = cu_ref[gg * _CH + e]
            pltpu.make_async_copy(
                utab_ref.at[:, :, pl.ds(pl.multiple_of(cu * 128, 128), 128)],
                ubuf.at[slot, :, :, pl.ds(e * 128, 128)],
                usem.at[slot]).start()
            ci = ci_ref[gg * _CH + e]
            pltpu.make_async_copy(
                itab_ref.at[:, :, pl.ds(pl.multiple_of(ci * 128, 128), 128)],
                ibuf.at[slot, :, :, pl.ds(e * 128, 128)],
                isem.at[slot]).start()

    @pl.when(g == 0)
    def _():
        issue(0, 0)

    slot = lax.rem(g, 2)

    @pl.when(g + 1 < nch)
    def _():
        issue(lax.rem(g + 1, 2), g + 1)

    pltpu.make_async_copy(
        utab_ref.at[:, :, pl.ds(0, _CH * 128)], ubuf.at[slot],
        usem.at[slot]).wait()
    pltpu.make_async_copy(
        itab_ref.at[:, :, pl.ds(0, _CH * 128)], ibuf.at[slot],
        isem.at[slot]).wait()

    lane = lax.broadcasted_iota(jnp.int32, (_CH, 128), 1)
    uoneh = (lane == mu_ref[...]).astype(jnp.float32)
    ioneh = (lane == mi_ref[...]).astype(jnp.float32)
    ub3 = ubuf[slot].reshape(_D, _CH, 128)
    ib3 = ibuf[slot].reshape(_D, _CH, 128)
    u64 = jnp.sum(ub3 * uoneh[None, :, :], axis=-1)
    i64 = jnp.sum(ib3 * ioneh[None, :, :], axis=-1)
    xT = jnp.concatenate([u64, i64], axis=0)

    out_ref[...] = _mlp_from_xT(
        xT, w1_ref[...], b1_ref[...], s1_ref[...], t1_ref[...],
        w2_ref[...], b2_ref[...], s2_ref[...], t2_ref[...],
        w3_ref[...], b3_ref[...], s3_ref[...], t3_ref[...],
        wo_ref[...], bo_ref[...])


def _fullp(shape):
    return pl.BlockSpec(shape, lambda i, *_: tuple(0 for _ in shape))


_tc_call = pl.pallas_call(
    _tc_body,
    grid_spec=pltpu.PrefetchScalarGridSpec(
        num_scalar_prefetch=2,
        grid=(_BTC // _CH,),
        in_specs=[
            pl.BlockSpec((_CH, 1), lambda i, *_: (i, 0)),
            pl.BlockSpec((_CH, 1), lambda i, *_: (i, 0)),
            pl.BlockSpec(memory_space=pl.ANY),
            pl.BlockSpec(memory_space=pl.ANY),
            _fullp((2 * _D, 128)), _fullp((1, 128)), _fullp((1, 128)),
            _fullp((1, 128)),
            _fullp((128, 64)), _fullp((1, 64)), _fullp((1, 64)),
            _fullp((1, 64)),
            _fullp((64, 32)), _fullp((1, 32)), _fullp((1, 32)),
            _fullp((1, 32)),
            _fullp((1, 32)), _fullp((1, 1)),
        ],
        out_specs=pl.BlockSpec((_CH, 1), lambda i, *_: (i, 0)),
        scratch_shapes=[
            pltpu.VMEM((2, 8, 8, _CH * 128), jnp.float32),
            pltpu.VMEM((2, 8, 8, _CH * 128), jnp.float32),
            pltpu.SemaphoreType.DMA((2,)),
            pltpu.SemaphoreType.DMA((2,)),
        ],
    ),
    out_shape=jax.ShapeDtypeStruct((_BTC, 1), jnp.float32),
)


def kernel(inputs, user_emb, item_emb, W1, b1, g1, be1, m1, v1,
           W2, b2, g2, be2, m2, v2, W3, b3, g3, be3, m3, v3, Wout, bout):
    uidx = inputs[:, 0]
    iidx = inputs[:, 1]
    utab3 = user_emb.T.reshape(8, 8, _V)
    itab3 = item_emb.T.reshape(8, 8, _V)

    def fold(g, be, m, v):
        s = g * lax.rsqrt(v + _EPS)
        return (s[None, :], (be - m * s)[None, :])

    s1, t1 = fold(g1, be1, m1, v1)
    s2, t2 = fold(g2, be2, m2, v2)
    s3, t3 = fold(g3, be3, m3, v3)
    wargs = (W1, b1[None, :], s1, t1,
             W2, b2[None, :], s2, t2,
             W3, b3[None, :], s3, t3,
             Wout.T, bout[None, :])

    xT_sc = _make_sc_gather()(uidx[:_S], iidx[:_S], utab3, itab3)
    out_sc = _mlp_call(xT_sc, *wargs)

    cu = (uidx[_S:] >> 7).astype(jnp.int32)
    mu = (uidx[_S:] & 127).astype(jnp.int32)
    ci = (iidx[_S:] >> 7).astype(jnp.int32)
    mi = (iidx[_S:] & 127).astype(jnp.int32)
    out_tc = _tc_call(cu, ci, mu[:, None], mi[:, None], utab3, itab3, *wargs)

    return jnp.concatenate([out_sc, out_tc], axis=0)
